# baseline (speedup 1.0000x reference)
import functools

import jax
import jax.numpy as jnp
from jax import lax
from jax.experimental import pallas as pl
from jax.experimental.pallas import tpu as pltpu
from jax.experimental.pallas import tpu_sc as plsc

N = 10000
E = 320000
D = 128

NC = 2
NS = 16
L = 16
NW = NC * NS
EPT = E // NW
EPTP = 10240
C = 80
CH = EPTP // C
NPAD = 10240
SL = NPAD // NS
NR = 10240
RPT = NR // NS


def _deg_body(dst_hbm, degpart_hbm, dstv, dtab, blk, outv, stg):
    cid = lax.axis_index("c")
    sid = lax.axis_index("s")
    wid = cid * NS + sid

    def zero_body(i, _):
        dtab[pl.ds(i * L, L)] = jnp.zeros((L,), jnp.float32)
        return 0

    lax.fori_loop(0, NPAD // L, zero_body, 0)

    pltpu.sync_copy(dst_hbm.at[wid], dstv)

    ones = jnp.ones((L,), jnp.float32)

    def acc_body(i, _):
        idx = dstv[pl.ds(i * L, L)]
        plsc.addupdate_scatter(dtab, [idx], ones)
        return 0

    lax.fori_loop(0, EPT // L, acc_body, 0)

    pltpu.sync_copy(dtab, stg.at[sid])
    plsc.subcore_barrier()
    for t in range(NS):
        pltpu.sync_copy(stg.at[t, pl.ds(sid * SL, SL)], blk.at[t])

    def red_body(k, _):
        v = blk[0, pl.ds(k * L, L)]
        for t in range(1, NS):
            v = v + blk[t, pl.ds(k * L, L)]
        outv[pl.ds(k * L, L)] = v
        return 0

    lax.fori_loop(0, SL // L, red_body, 0)
    pltpu.sync_copy(outv, degpart_hbm.at[cid, pl.ds(sid * SL, SL)])


def _deg_kernel(dst2d):
    mesh = plsc.VectorSubcoreMesh(core_axis_name="c", subcore_axis_name="s",
                                  num_cores=NC, num_subcores=NS)
    return pl.kernel(
        _deg_body,
        out_type=jax.ShapeDtypeStruct((NC, NPAD), jnp.float32),
        mesh=mesh,
        compiler_params=pltpu.CompilerParams(needs_layout_passes=False),
        scratch_types=[
            pltpu.VMEM((EPT,), jnp.int32),
            pltpu.VMEM((NPAD,), jnp.float32),
            pltpu.VMEM((NS, SL), jnp.float32),
            pltpu.VMEM((SL,), jnp.float32),
            pltpu.VMEM_SHARED((NS, NPAD), jnp.float32),
        ],
    )(dst2d)


def _mm_body(x_ref, w_ref, b_ref, degt_ref, g_ref):
    h = jnp.dot(x_ref[...], w_ref[...].T, preferred_element_type=jnp.float32)
    h = h + b_ref[...]
    deg = degt_ref[:, 0] + degt_ref[:, 1] + 1.0
    norm = lax.rsqrt(deg)
    g_ref[...] = h * norm[:, None]


def _mm_kernel(x, w, b2d, degt):
    BR = 400
    grid = (N // BR,)
    return pl.pallas_call(
        _mm_body,
        grid=grid,
        in_specs=[
            pl.BlockSpec((BR, D), lambda i: (i, 0)),
            pl.BlockSpec((D, D), lambda i: (0, 0)),
            pl.BlockSpec((1, D), lambda i: (0, 0)),
            pl.BlockSpec((BR, NC), lambda i: (i, 0)),
        ],
        out_specs=pl.BlockSpec((BR, D), lambda i: (i, 0)),
        out_shape=jax.ShapeDtypeStruct((NR, D), jnp.float32),
    )(x, w, b2d, degt)


def _scatter_body(g_hbm, zero_hbm, src_hbm, dst_hbm, part_hbm,
                  sidx, didx, rows0, acc, sem0):
    cid = lax.axis_index("c")
    sid = lax.axis_index("s")
    wid = cid * NS + sid
    rsl = pl.ds(sid * RPT, RPT)

    @pl.when(cid == 0)
    def _():
        pltpu.sync_copy(g_hbm.at[rsl], acc.at[rsl])

    @pl.when(cid == 1)
    def _():
        pltpu.sync_copy(zero_hbm.at[rsl], acc.at[rsl])

    pltpu.sync_copy(src_hbm.at[wid], sidx)
    pltpu.sync_copy(dst_hbm.at[wid], didx)
    plsc.subcore_barrier()

    def chunk_body(j, _):
        pltpu.async_copy(g_hbm.at[sidx.at[j]], rows0, sem0).wait()
        pltpu.sync_copy(rows0, acc.at[didx.at[j]], add=True)
        return 0

    lax.fori_loop(0, CH, chunk_body, 0)

    plsc.subcore_barrier()
    pltpu.sync_copy(acc.at[rsl], part_hbm.at[cid, rsl])


def _scatter_kernel(g, zero, src3, dst3):
    mesh = plsc.VectorSubcoreMesh(core_axis_name="c", subcore_axis_name="s",
                                  num_cores=NC, num_subcores=NS)
    return pl.kernel(
        _scatter_body,
        out_type=jax.ShapeDtypeStruct((NC, NR, D), jnp.float32),
        mesh=mesh,
        scratch_types=[
            pltpu.VMEM((CH, C), jnp.int32),
            pltpu.VMEM((CH, C), jnp.int32),
            pltpu.VMEM((C, D), jnp.float32),
            pltpu.VMEM_SHARED((NR, D), jnp.float32),
            pltpu.SemaphoreType.DMA,
        ],
    )(g, zero, src3, dst3)


def _fin_body(part_ref, degt_ref, out_ref):
    deg = degt_ref[:, 0] + degt_ref[:, 1] + 1.0
    norm = lax.rsqrt(deg)
    out_ref[...] = (part_ref[0] + part_ref[1]) * norm[:, None]


def _fin_kernel(part, degt):
    BR = 400
    grid = (N // BR,)
    return pl.pallas_call(
        _fin_body,
        grid=grid,
        in_specs=[
            pl.BlockSpec((NC, BR, D), lambda i: (0, i, 0)),
            pl.BlockSpec((BR, NC), lambda i: (i, 0)),
        ],
        out_specs=pl.BlockSpec((BR, D), lambda i: (i, 0)),
        out_shape=jax.ShapeDtypeStruct((N, D), jnp.float32),
    )(part, degt)


@jax.jit
def kernel(x, edge_index, W, b):
    src = edge_index[0]
    dst = edge_index[1]
    dst2d = dst.reshape(NW, EPT)
    pad_src = jnp.zeros((NW, EPTP - EPT), jnp.int32)
    pad_dst = jnp.broadcast_to(
        N + jnp.arange(EPTP - EPT, dtype=jnp.int32) % (NR - N),
        (NW, EPTP - EPT))
    src3 = jnp.concatenate([src.reshape(NW, EPT), pad_src],
                           axis=1).reshape(NW, CH, C)
    dst3 = jnp.concatenate([dst.reshape(NW, EPT), pad_dst],
                           axis=1).reshape(NW, CH, C)
    b2d = b.reshape(1, D)
    zero = jnp.zeros((NR, D), jnp.float32)

    degpart = _deg_kernel(dst2d)
    degt = degpart.T
    g = _mm_kernel(x, W, b2d, degt)
    part = _scatter_kernel(g, zero, src3, dst3)
    out = _fin_kernel(part, degt)
    return out


# speedup vs baseline: 1.9270x; 1.9270x over previous
import functools

import jax
import jax.numpy as jnp
from jax import lax
from jax.experimental import pallas as pl
from jax.experimental.pallas import tpu as pltpu
from jax.experimental.pallas import tpu_sc as plsc

N = 10000
E = 320000
D = 128

NC = 2
NS = 16
L = 16
NW = NC * NS
EPT = E // NW
EPTP = 10240
C = 80
CH = EPTP // C
NPAD = 10240
SL = NPAD // NS
NR = 10240
RPT = NR // NS


def _deg_body(dst_hbm, degpart_hbm, dstv, dtab, blk, outv, stg):
    cid = lax.axis_index("c")
    sid = lax.axis_index("s")
    wid = cid * NS + sid

    def zero_body(i, _):
        dtab[pl.ds(i * L, L)] = jnp.zeros((L,), jnp.float32)
        return 0

    lax.fori_loop(0, NPAD // L, zero_body, 0)

    pltpu.sync_copy(dst_hbm.at[wid], dstv)

    ones = jnp.ones((L,), jnp.float32)

    def acc_body(i, _):
        idx = dstv[pl.ds(i * L, L)]
        plsc.addupdate_scatter(dtab, [idx], ones)
        return 0

    lax.fori_loop(0, EPT // L, acc_body, 0)

    pltpu.sync_copy(dtab, stg.at[sid])
    plsc.subcore_barrier()
    for t in range(NS):
        pltpu.sync_copy(stg.at[t, pl.ds(sid * SL, SL)], blk.at[t])

    def red_body(k, _):
        v = blk[0, pl.ds(k * L, L)]
        for t in range(1, NS):
            v = v + blk[t, pl.ds(k * L, L)]
        outv[pl.ds(k * L, L)] = v
        return 0

    lax.fori_loop(0, SL // L, red_body, 0)
    pltpu.sync_copy(outv, degpart_hbm.at[cid, pl.ds(sid * SL, SL)])


def _deg_kernel(dst2d):
    mesh = plsc.VectorSubcoreMesh(core_axis_name="c", subcore_axis_name="s",
                                  num_cores=NC, num_subcores=NS)
    return pl.kernel(
        _deg_body,
        out_type=jax.ShapeDtypeStruct((NC, NPAD), jnp.float32),
        mesh=mesh,
        compiler_params=pltpu.CompilerParams(needs_layout_passes=False),
        scratch_types=[
            pltpu.VMEM((EPT,), jnp.int32),
            pltpu.VMEM((NPAD,), jnp.float32),
            pltpu.VMEM((NS, SL), jnp.float32),
            pltpu.VMEM((SL,), jnp.float32),
            pltpu.VMEM_SHARED((NS, NPAD), jnp.float32),
        ],
    )(dst2d)


def _mm_body(x_ref, w_ref, b_ref, degt_ref, g_ref):
    h = jnp.dot(x_ref[...], w_ref[...].T, preferred_element_type=jnp.float32)
    h = h + b_ref[...]
    deg = degt_ref[:, 0] + degt_ref[:, 1] + 1.0
    norm = lax.rsqrt(deg)
    g_ref[...] = h * norm[:, None]


def _mm_kernel(x, w, b2d, degt):
    BR = 400
    grid = (N // BR,)
    return pl.pallas_call(
        _mm_body,
        grid=grid,
        in_specs=[
            pl.BlockSpec((BR, D), lambda i: (i, 0)),
            pl.BlockSpec((D, D), lambda i: (0, 0)),
            pl.BlockSpec((1, D), lambda i: (0, 0)),
            pl.BlockSpec((BR, NC), lambda i: (i, 0)),
        ],
        out_specs=pl.BlockSpec((BR, D), lambda i: (i, 0)),
        out_shape=jax.ShapeDtypeStruct((NR, D), jnp.float32),
    )(x, w, b2d, degt)


def _scatter_body(g_hbm, zero_hbm, src_hbm, dst_hbm, part_hbm,
                  sidx, didx, rows0, acc, sem0):
    cid = lax.axis_index("c")
    sid = lax.axis_index("s")
    wid = cid * NS + sid
    rsl = pl.ds(sid * RPT, RPT)

    @pl.when(cid == 0)
    def _():
        pltpu.sync_copy(g_hbm.at[rsl], acc.at[rsl])

    @pl.when(cid == 1)
    def _():
        pltpu.sync_copy(zero_hbm.at[rsl], acc.at[rsl])

    pltpu.sync_copy(src_hbm.at[wid], sidx)
    pltpu.sync_copy(dst_hbm.at[wid], didx)
    plsc.subcore_barrier()

    def chunk_body(j, _):
        pltpu.async_copy(g_hbm.at[sidx.at[j]], rows0, sem0).wait()
        pltpu.sync_copy(rows0, acc.at[didx.at[j]], add=True)
        return 0

    lax.fori_loop(0, CH, chunk_body, 0)

    plsc.subcore_barrier()
    pltpu.sync_copy(acc.at[rsl], part_hbm.at[cid, rsl])


def _scatter_kernel(g, zero, src3, dst3):
    mesh = plsc.VectorSubcoreMesh(core_axis_name="c", subcore_axis_name="s",
                                  num_cores=NC, num_subcores=NS)
    return pl.kernel(
        _scatter_body,
        out_type=jax.ShapeDtypeStruct((NC, NR, D), jnp.float32),
        mesh=mesh,
        scratch_types=[
            pltpu.VMEM((CH, C), jnp.int32),
            pltpu.VMEM((CH, C), jnp.int32),
            pltpu.VMEM((C, D), jnp.float32),
            pltpu.VMEM_SHARED((NR, D), jnp.float32),
            pltpu.SemaphoreType.DMA,
        ],
    )(g, zero, src3, dst3)


def _fin_body(part_ref, degt_ref, out_ref):
    deg = degt_ref[:, 0] + degt_ref[:, 1] + 1.0
    norm = lax.rsqrt(deg)
    out_ref[...] = (part_ref[0] + part_ref[1]) * norm[:, None]


def _fin_kernel(part, degt):
    BR = 400
    grid = (N // BR,)
    return pl.pallas_call(
        _fin_body,
        grid=grid,
        in_specs=[
            pl.BlockSpec((NC, BR, D), lambda i: (0, i, 0)),
            pl.BlockSpec((BR, NC), lambda i: (i, 0)),
        ],
        out_specs=pl.BlockSpec((BR, D), lambda i: (i, 0)),
        out_shape=jax.ShapeDtypeStruct((N, D), jnp.float32),
    )(part, degt)


@jax.jit
def kernel(x, edge_index, W, b):
    src = edge_index[0]
    dst = edge_index[1]
    dst2d = dst.reshape(NW, EPT)
    pad_src = jnp.broadcast_to(
        jnp.arange(EPTP - EPT, dtype=jnp.int32), (NW, EPTP - EPT))
    pad_dst = jnp.broadcast_to(
        N + jnp.arange(EPTP - EPT, dtype=jnp.int32) % (NR - N),
        (NW, EPTP - EPT))
    src3 = jnp.concatenate([src.reshape(NW, EPT), pad_src],
                           axis=1).reshape(NW, CH, C)
    dst3 = jnp.concatenate([dst.reshape(NW, EPT), pad_dst],
                           axis=1).reshape(NW, CH, C)
    b2d = b.reshape(1, D)
    zero = jnp.zeros((NR, D), jnp.float32)

    degpart = _deg_kernel(dst2d)
    degt = degpart.T
    g = _mm_kernel(x, W, b2d, degt)
    part = _scatter_kernel(g, zero, src3, dst3)
    out = _fin_kernel(part, degt)
    return out


# speedup vs baseline: 2.9509x; 1.5314x over previous
import functools

import jax
import jax.numpy as jnp
from jax import lax
from jax.experimental import pallas as pl
from jax.experimental.pallas import tpu as pltpu
from jax.experimental.pallas import tpu_sc as plsc

N = 10000
E = 320000
D = 128

NC = 2
NS = 16
L = 16
NW = NC * NS
EPT = E // NW
EPTP = 10240
C = 128
CH = EPTP // C
NPAD = 10240
SL = NPAD // NS
NR = 10240
RPT = NR // NS


def _deg_body(dst_hbm, degpart_hbm, dstv, dtab, blk, outv, stg):
    cid = lax.axis_index("c")
    sid = lax.axis_index("s")
    wid = cid * NS + sid

    def zero_body(i, _):
        dtab[pl.ds(i * L, L)] = jnp.zeros((L,), jnp.float32)
        return 0

    lax.fori_loop(0, NPAD // L, zero_body, 0)

    pltpu.sync_copy(dst_hbm.at[wid], dstv)

    ones = jnp.ones((L,), jnp.float32)

    def acc_body(i, _):
        idx = dstv[pl.ds(i * L, L)]
        plsc.addupdate_scatter(dtab, [idx], ones)
        return 0

    lax.fori_loop(0, EPT // L, acc_body, 0)

    pltpu.sync_copy(dtab, stg.at[sid])
    plsc.subcore_barrier()
    for t in range(NS):
        pltpu.sync_copy(stg.at[t, pl.ds(sid * SL, SL)], blk.at[t])

    def red_body(k, _):
        v = blk[0, pl.ds(k * L, L)]
        for t in range(1, NS):
            v = v + blk[t, pl.ds(k * L, L)]
        outv[pl.ds(k * L, L)] = v
        return 0

    lax.fori_loop(0, SL // L, red_body, 0)
    pltpu.sync_copy(outv, degpart_hbm.at[cid, pl.ds(sid * SL, SL)])


def _deg_kernel(dst2d):
    mesh = plsc.VectorSubcoreMesh(core_axis_name="c", subcore_axis_name="s",
                                  num_cores=NC, num_subcores=NS)
    return pl.kernel(
        _deg_body,
        out_type=jax.ShapeDtypeStruct((NC, NPAD), jnp.float32),
        mesh=mesh,
        compiler_params=pltpu.CompilerParams(needs_layout_passes=False),
        scratch_types=[
            pltpu.VMEM((EPT,), jnp.int32),
            pltpu.VMEM((NPAD,), jnp.float32),
            pltpu.VMEM((NS, SL), jnp.float32),
            pltpu.VMEM((SL,), jnp.float32),
            pltpu.VMEM_SHARED((NS, NPAD), jnp.float32),
        ],
    )(dst2d)


def _mm_body(x_ref, w_ref, b_ref, degt_ref, g_ref):
    h = jnp.dot(x_ref[...], w_ref[...].T, preferred_element_type=jnp.float32)
    h = h + b_ref[...]
    deg = degt_ref[:, 0] + degt_ref[:, 1] + 1.0
    norm = lax.rsqrt(deg)
    g_ref[...] = h * norm[:, None]


def _mm_kernel(x, w, b2d, degt):
    BR = 400
    grid = (N // BR,)
    return pl.pallas_call(
        _mm_body,
        grid=grid,
        in_specs=[
            pl.BlockSpec((BR, D), lambda i: (i, 0)),
            pl.BlockSpec((D, D), lambda i: (0, 0)),
            pl.BlockSpec((1, D), lambda i: (0, 0)),
            pl.BlockSpec((BR, NC), lambda i: (i, 0)),
        ],
        out_specs=pl.BlockSpec((BR, D), lambda i: (i, 0)),
        out_shape=jax.ShapeDtypeStruct((NR, D), jnp.float32),
    )(x, w, b2d, degt)


def _scatter_body(g_hbm, zero_hbm, src_hbm, dst_hbm, part_hbm,
                  sidx, dbuf0, dbuf1, rows0, rows1, acc,
                  sem0, sem1, semd0, semd1):
    cid = lax.axis_index("c")
    sid = lax.axis_index("s")
    wid = cid * NS + sid
    rsl = pl.ds(sid * RPT, RPT)

    @pl.when(cid == 0)
    def _():
        pltpu.sync_copy(g_hbm.at[rsl], acc.at[rsl])

    @pl.when(cid == 1)
    def _():
        pltpu.sync_copy(zero_hbm.at[rsl], acc.at[rsl])

    pltpu.sync_copy(src_hbm.at[wid], sidx)

    def gather(j, rows, sem):
        return pltpu.make_async_copy(
            g_hbm.at[sidx.at[pl.ds(j * C, C)]], rows, sem)

    def dfetch(j, dbuf, semd):
        return pltpu.make_async_copy(dst_hbm.at[wid, j], dbuf, semd)

    dfetch(0, dbuf0, semd0).start()
    gather(0, rows0, sem0).start()
    plsc.subcore_barrier()

    def pair_body(jj, _):
        j0 = jj * 2
        j1 = j0 + 1
        gather(j1, rows1, sem1).start()
        dfetch(j1, dbuf1, semd1).start()
        gather(j0, rows0, sem0).wait()
        dfetch(j0, dbuf0, semd0).wait()
        pltpu.sync_copy(rows0, acc.at[dbuf0], add=True)

        @pl.when(jj < CH // 2 - 1)
        def _():
            gather(j0 + 2, rows0, sem0).start()
            dfetch(j0 + 2, dbuf0, semd0).start()

        gather(j1, rows1, sem1).wait()
        dfetch(j1, dbuf1, semd1).wait()
        pltpu.sync_copy(rows1, acc.at[dbuf1], add=True)
        return 0

    lax.fori_loop(0, CH // 2, pair_body, 0)

    plsc.subcore_barrier()
    pltpu.sync_copy(acc.at[rsl], part_hbm.at[cid, rsl])


def _scatter_kernel(g, zero, src3, dst3):
    mesh = plsc.VectorSubcoreMesh(core_axis_name="c", subcore_axis_name="s",
                                  num_cores=NC, num_subcores=NS)
    return pl.kernel(
        _scatter_body,
        out_type=jax.ShapeDtypeStruct((NC, NR, D), jnp.float32),
        mesh=mesh,
        scratch_types=[
            pltpu.VMEM((EPTP,), jnp.int32),
            pltpu.VMEM((C,), jnp.int32),
            pltpu.VMEM((C,), jnp.int32),
            pltpu.VMEM((C, D), jnp.float32),
            pltpu.VMEM((C, D), jnp.float32),
            pltpu.VMEM_SHARED((NR, D), jnp.float32),
            pltpu.SemaphoreType.DMA,
            pltpu.SemaphoreType.DMA,
            pltpu.SemaphoreType.DMA,
            pltpu.SemaphoreType.DMA,
        ],
    )(g, zero, src3, dst3)


def _fin_body(part_ref, degt_ref, out_ref):
    deg = degt_ref[:, 0] + degt_ref[:, 1] + 1.0
    norm = lax.rsqrt(deg)
    out_ref[...] = (part_ref[0] + part_ref[1]) * norm[:, None]


def _fin_kernel(part, degt):
    BR = 400
    grid = (N // BR,)
    return pl.pallas_call(
        _fin_body,
        grid=grid,
        in_specs=[
            pl.BlockSpec((NC, BR, D), lambda i: (0, i, 0)),
            pl.BlockSpec((BR, NC), lambda i: (i, 0)),
        ],
        out_specs=pl.BlockSpec((BR, D), lambda i: (i, 0)),
        out_shape=jax.ShapeDtypeStruct((N, D), jnp.float32),
    )(part, degt)


@jax.jit
def kernel(x, edge_index, W, b):
    src = edge_index[0]
    dst = edge_index[1]
    dst2d = dst.reshape(NW, EPT)
    pad_src = jnp.broadcast_to(
        jnp.arange(EPTP - EPT, dtype=jnp.int32), (NW, EPTP - EPT))
    pad_dst = jnp.broadcast_to(
        N + jnp.arange(EPTP - EPT, dtype=jnp.int32) % (NR - N),
        (NW, EPTP - EPT))
    src3 = jnp.concatenate([src.reshape(NW, EPT), pad_src], axis=1)
    dst3 = jnp.concatenate([dst.reshape(NW, EPT), pad_dst],
                           axis=1).reshape(NW, CH, C)
    b2d = b.reshape(1, D)
    zero = jnp.zeros((NR, D), jnp.float32)

    degpart = _deg_kernel(dst2d)
    degt = degpart.T
    g = _mm_kernel(x, W, b2d, degt)
    part = _scatter_kernel(g, zero, src3, dst3)
    out = _fin_kernel(part, degt)
    return out


# speedup vs baseline: 3.1381x; 1.0634x over previous
import functools

import jax
import jax.numpy as jnp
from jax import lax
from jax.experimental import pallas as pl
from jax.experimental.pallas import tpu as pltpu
from jax.experimental.pallas import tpu_sc as plsc

N = 10000
E = 320000
D = 128

NC = 2
NS = 16
L = 16
NW = NC * NS
EPT = E // NW
EPTP = 10240
C = 64
CH = EPTP // C
NBUF = 4
NPAD = 10240
SL = NPAD // NS
NR = 10240
RPT = NR // NS


def _deg_body(dst_hbm, degpart_hbm, dstv, dtab, blk, outv, stg):
    cid = lax.axis_index("c")
    sid = lax.axis_index("s")
    wid = cid * NS + sid

    def zero_body(i, _):
        dtab[pl.ds(i * L, L)] = jnp.zeros((L,), jnp.float32)
        return 0

    lax.fori_loop(0, NPAD // L, zero_body, 0)

    pltpu.sync_copy(dst_hbm.at[wid], dstv)

    ones = jnp.ones((L,), jnp.float32)

    def acc_body(i, _):
        idx = dstv[pl.ds(i * L, L)]
        plsc.addupdate_scatter(dtab, [idx], ones)
        return 0

    lax.fori_loop(0, EPT // L, acc_body, 0)

    pltpu.sync_copy(dtab, stg.at[sid])
    plsc.subcore_barrier()
    for t in range(NS):
        pltpu.sync_copy(stg.at[t, pl.ds(sid * SL, SL)], blk.at[t])

    def red_body(k, _):
        v = blk[0, pl.ds(k * L, L)]
        for t in range(1, NS):
            v = v + blk[t, pl.ds(k * L, L)]
        outv[pl.ds(k * L, L)] = v
        return 0

    lax.fori_loop(0, SL // L, red_body, 0)
    pltpu.sync_copy(outv, degpart_hbm.at[cid, pl.ds(sid * SL, SL)])


def _deg_kernel(dst2d):
    mesh = plsc.VectorSubcoreMesh(core_axis_name="c", subcore_axis_name="s",
                                  num_cores=NC, num_subcores=NS)
    return pl.kernel(
        _deg_body,
        out_type=jax.ShapeDtypeStruct((NC, NPAD), jnp.float32),
        mesh=mesh,
        compiler_params=pltpu.CompilerParams(needs_layout_passes=False),
        scratch_types=[
            pltpu.VMEM((EPT,), jnp.int32),
            pltpu.VMEM((NPAD,), jnp.float32),
            pltpu.VMEM((NS, SL), jnp.float32),
            pltpu.VMEM((SL,), jnp.float32),
            pltpu.VMEM_SHARED((NS, NPAD), jnp.float32),
        ],
    )(dst2d)


MMBR = 400


def _mm_body(x_ref, w_ref, b_ref, degt_ref, g_ref):
    h = jnp.dot(x_ref[...], w_ref[...].T, preferred_element_type=jnp.float32)
    h = h + b_ref[...]
    deg = degt_ref[:, 0] + degt_ref[:, 1] + 1.0
    norm = lax.rsqrt(deg)
    g_ref[...] = h * norm[:, None]


def _mm_kernel(x, w, b2d, degt):
    grid = (N // MMBR,)
    return pl.pallas_call(
        _mm_body,
        grid=grid,
        in_specs=[
            pl.BlockSpec((MMBR, D), lambda i: (i, 0)),
            pl.BlockSpec((D, D), lambda i: (0, 0)),
            pl.BlockSpec((1, D), lambda i: (0, 0)),
            pl.BlockSpec((MMBR, NC), lambda i: (i, 0)),
        ],
        out_specs=pl.BlockSpec((MMBR, D), lambda i: (i, 0)),
        out_shape=jax.ShapeDtypeStruct((NR, D), jnp.float32),
    )(x, w, b2d, degt)


def _scatter_body(g_hbm, zero_hbm, src_hbm, dst_hbm, part_hbm,
                  sidx, db0, db1, db2, db3, rw0, rw1, rw2, rw3, acc,
                  sg0, sg1, sg2, sg3, sd0, sd1, sd2, sd3):
    cid = lax.axis_index("c")
    sid = lax.axis_index("s")
    wid = cid * NS + sid
    rsl = pl.ds(sid * RPT, RPT)
    dbufs = (db0, db1, db2, db3)
    rows = (rw0, rw1, rw2, rw3)
    gsems = (sg0, sg1, sg2, sg3)
    dsems = (sd0, sd1, sd2, sd3)

    @pl.when(cid == 0)
    def _():
        pltpu.sync_copy(g_hbm.at[rsl], acc.at[rsl])

    @pl.when(cid == 1)
    def _():
        pltpu.sync_copy(zero_hbm.at[rsl], acc.at[rsl])

    pltpu.sync_copy(src_hbm.at[wid], sidx)

    def gather(j, k):
        return pltpu.make_async_copy(
            g_hbm.at[sidx.at[pl.ds(j * C, C)]], rows[k], gsems[k])

    def dfetch(j, k):
        return pltpu.make_async_copy(dst_hbm.at[wid, j], dbufs[k], dsems[k])

    for k in range(NBUF):
        dfetch(k, k).start()
        gather(k, k).start()
    plsc.subcore_barrier()

    def ring_body(q, _):
        j = q * NBUF
        for k in range(NBUF):
            jk = j + k
            gather(jk, k).wait()
            dfetch(jk, k).wait()
            pltpu.sync_copy(rows[k], acc.at[dbufs[k]], add=True)

            @pl.when(jk + NBUF < CH)
            def _():
                gather(jk + NBUF, k).start()
                dfetch(jk + NBUF, k).start()
        return 0

    lax.fori_loop(0, CH // NBUF, ring_body, 0)

    plsc.subcore_barrier()
    pltpu.sync_copy(acc.at[rsl], part_hbm.at[cid, rsl])


def _scatter_kernel(g, zero, src3, dst3):
    mesh = plsc.VectorSubcoreMesh(core_axis_name="c", subcore_axis_name="s",
                                  num_cores=NC, num_subcores=NS)
    return pl.kernel(
        _scatter_body,
        out_type=jax.ShapeDtypeStruct((NC, NR, D), jnp.float32),
        mesh=mesh,
        scratch_types=(
            [pltpu.VMEM((EPTP,), jnp.int32)]
            + [pltpu.VMEM((C,), jnp.int32) for _ in range(NBUF)]
            + [pltpu.VMEM((C, D), jnp.float32) for _ in range(NBUF)]
            + [pltpu.VMEM_SHARED((NR, D), jnp.float32)]
            + [pltpu.SemaphoreType.DMA for _ in range(2 * NBUF)]
        ),
    )(g, zero, src3, dst3)


def _fin_body(part_ref, degt_ref, out_ref):
    deg = degt_ref[:, 0] + degt_ref[:, 1] + 1.0
    norm = lax.rsqrt(deg)
    p = part_ref[0] + part_ref[1]
    out_ref[...] = p * norm[:, None]


def _fin_kernel(part, degt):
    grid = (N // MMBR,)
    return pl.pallas_call(
        _fin_body,
        grid=grid,
        in_specs=[
            pl.BlockSpec((NC, MMBR, D), lambda i: (0, i, 0)),
            pl.BlockSpec((MMBR, NC), lambda i: (i, 0)),
        ],
        out_specs=pl.BlockSpec((MMBR, D), lambda i: (i, 0)),
        out_shape=jax.ShapeDtypeStruct((N, D), jnp.float32),
    )(part, degt)


@jax.jit
def kernel(x, edge_index, W, b):
    src = edge_index[0]
    dst = edge_index[1]
    dst2d = dst.reshape(NW, EPT)
    pad_src = jnp.broadcast_to(
        jnp.arange(EPTP - EPT, dtype=jnp.int32), (NW, EPTP - EPT))
    pad_dst = jnp.broadcast_to(
        N + jnp.arange(EPTP - EPT, dtype=jnp.int32) % (NR - N),
        (NW, EPTP - EPT))
    src3 = jnp.concatenate([src.reshape(NW, EPT), pad_src], axis=1)
    dst3 = jnp.concatenate([dst.reshape(NW, EPT), pad_dst],
                           axis=1).reshape(NW, CH, C)
    b2d = b.reshape(1, D)
    zero = jnp.zeros((NR, D), jnp.float32)

    degpart = _deg_kernel(dst2d)
    degt = degpart.T
    g = _mm_kernel(x, W, b2d, degt)
    part = _scatter_kernel(g, zero, src3, dst3)
    out = _fin_kernel(part, degt)
    return out


# speedup vs baseline: 3.1883x; 1.0160x over previous
import functools

import jax
import jax.numpy as jnp
from jax import lax
from jax.experimental import pallas as pl
from jax.experimental.pallas import tpu as pltpu
from jax.experimental.pallas import tpu_sc as plsc

N = 10000
E = 320000
D = 128

NC = 2
NS = 16
L = 16
NW = NC * NS
EPT = E // NW
EPTP = EPT
C = 40
CH = EPTP // C
NBUF = 5
NPAD = 10240
SL = NPAD // NS
NR = 10240
RPT = NR // NS


def _deg_body(dst_hbm, degpart_hbm, dstv, dtab, blk, outv, stg):
    cid = lax.axis_index("c")
    sid = lax.axis_index("s")
    wid = cid * NS + sid

    def zero_body(i, _):
        dtab[pl.ds(i * L, L)] = jnp.zeros((L,), jnp.float32)
        return 0

    lax.fori_loop(0, NPAD // L, zero_body, 0)

    pltpu.sync_copy(dst_hbm.at[wid], dstv)

    ones = jnp.ones((L,), jnp.float32)

    def acc_body(i, _):
        idx = dstv[pl.ds(i * L, L)]
        plsc.addupdate_scatter(dtab, [idx], ones)
        return 0

    lax.fori_loop(0, EPT // L, acc_body, 0)

    pltpu.sync_copy(dtab, stg.at[sid])
    plsc.subcore_barrier()
    for t in range(NS):
        pltpu.sync_copy(stg.at[t, pl.ds(sid * SL, SL)], blk.at[t])

    def red_body(k, _):
        v = blk[0, pl.ds(k * L, L)]
        for t in range(1, NS):
            v = v + blk[t, pl.ds(k * L, L)]
        outv[pl.ds(k * L, L)] = v
        return 0

    lax.fori_loop(0, SL // L, red_body, 0)
    pltpu.sync_copy(outv, degpart_hbm.at[cid, pl.ds(sid * SL, SL)])


def _deg_kernel(dst2d):
    mesh = plsc.VectorSubcoreMesh(core_axis_name="c", subcore_axis_name="s",
                                  num_cores=NC, num_subcores=NS)
    return pl.kernel(
        _deg_body,
        out_type=jax.ShapeDtypeStruct((NC, NPAD), jnp.float32),
        mesh=mesh,
        compiler_params=pltpu.CompilerParams(needs_layout_passes=False),
        scratch_types=[
            pltpu.VMEM((EPT,), jnp.int32),
            pltpu.VMEM((NPAD,), jnp.float32),
            pltpu.VMEM((NS, SL), jnp.float32),
            pltpu.VMEM((SL,), jnp.float32),
            pltpu.VMEM_SHARED((NS, NPAD), jnp.float32),
        ],
    )(dst2d)


MMBR = 400


def _mm_body(x_ref, w_ref, b_ref, degt_ref, g_ref):
    h = jnp.dot(x_ref[...], w_ref[...].T, preferred_element_type=jnp.float32)
    h = h + b_ref[...]
    deg = degt_ref[:, 0] + degt_ref[:, 1] + 1.0
    norm = lax.rsqrt(deg)
    g_ref[...] = h * norm[:, None]


def _mm_kernel(x, w, b2d, degt):
    grid = (N // MMBR,)
    return pl.pallas_call(
        _mm_body,
        grid=grid,
        in_specs=[
            pl.BlockSpec((MMBR, D), lambda i: (i, 0)),
            pl.BlockSpec((D, D), lambda i: (0, 0)),
            pl.BlockSpec((1, D), lambda i: (0, 0)),
            pl.BlockSpec((MMBR, NC), lambda i: (i, 0)),
        ],
        out_specs=pl.BlockSpec((MMBR, D), lambda i: (i, 0)),
        out_shape=jax.ShapeDtypeStruct((NR, D), jnp.float32),
    )(x, w, b2d, degt)


def _scatter_body(g_hbm, zero_hbm, src_hbm, dst_hbm, part_hbm,
                  sidx, db0, db1, db2, db3, db4, rw0, rw1, rw2, rw3, rw4, acc,
                  sg0, sg1, sg2, sg3, sg4, sd0, sd1, sd2, sd3, sd4):
    cid = lax.axis_index("c")
    sid = lax.axis_index("s")
    wid = cid * NS + sid
    rsl = pl.ds(sid * RPT, RPT)
    dbufs = (db0, db1, db2, db3, db4)
    rows = (rw0, rw1, rw2, rw3, rw4)
    gsems = (sg0, sg1, sg2, sg3, sg4)
    dsems = (sd0, sd1, sd2, sd3, sd4)

    @pl.when(cid == 0)
    def _():
        pltpu.sync_copy(g_hbm.at[rsl], acc.at[rsl])

    @pl.when(cid == 1)
    def _():
        pltpu.sync_copy(zero_hbm.at[rsl], acc.at[rsl])

    pltpu.sync_copy(src_hbm.at[wid], sidx)

    def gather(j, k):
        return pltpu.make_async_copy(
            g_hbm.at[sidx.at[pl.ds(j * C, C)]], rows[k], gsems[k])

    def dfetch(j, k):
        return pltpu.make_async_copy(dst_hbm.at[wid, j], dbufs[k], dsems[k])

    for k in range(NBUF):
        dfetch(k, k).start()
        gather(k, k).start()
    plsc.subcore_barrier()

    def ring_body(q, _):
        j = q * NBUF
        for k in range(NBUF):
            jk = j + k
            gather(jk, k).wait()
            dfetch(jk, k).wait()
            pltpu.sync_copy(rows[k], acc.at[dbufs[k]], add=True)

            @pl.when(jk + NBUF < CH)
            def _():
                gather(jk + NBUF, k).start()
                dfetch(jk + NBUF, k).start()
        return 0

    lax.fori_loop(0, CH // NBUF, ring_body, 0)

    plsc.subcore_barrier()
    pltpu.sync_copy(acc.at[rsl], part_hbm.at[cid, rsl])


def _scatter_kernel(g, zero, src3, dst3):
    mesh = plsc.VectorSubcoreMesh(core_axis_name="c", subcore_axis_name="s",
                                  num_cores=NC, num_subcores=NS)
    return pl.kernel(
        _scatter_body,
        out_type=jax.ShapeDtypeStruct((NC, NR, D), jnp.float32),
        mesh=mesh,
        scratch_types=(
            [pltpu.VMEM((EPTP,), jnp.int32)]
            + [pltpu.VMEM((C,), jnp.int32) for _ in range(NBUF)]
            + [pltpu.VMEM((C, D), jnp.float32) for _ in range(NBUF)]
            + [pltpu.VMEM_SHARED((NR, D), jnp.float32)]
            + [pltpu.SemaphoreType.DMA for _ in range(2 * NBUF)]
        ),
    )(g, zero, src3, dst3)


def _fin_body(part_ref, degt_ref, out_ref):
    deg = degt_ref[:, 0] + degt_ref[:, 1] + 1.0
    norm = lax.rsqrt(deg)
    p = part_ref[0] + part_ref[1]
    out_ref[...] = p * norm[:, None]


def _fin_kernel(part, degt):
    grid = (N // MMBR,)
    return pl.pallas_call(
        _fin_body,
        grid=grid,
        in_specs=[
            pl.BlockSpec((NC, MMBR, D), lambda i: (0, i, 0)),
            pl.BlockSpec((MMBR, NC), lambda i: (i, 0)),
        ],
        out_specs=pl.BlockSpec((MMBR, D), lambda i: (i, 0)),
        out_shape=jax.ShapeDtypeStruct((N, D), jnp.float32),
    )(part, degt)


@jax.jit
def kernel(x, edge_index, W, b):
    src = edge_index[0]
    dst = edge_index[1]
    dst2d = dst.reshape(NW, EPT)
    src3 = src.reshape(NW, EPT)
    dst3 = dst.reshape(NW, CH, C)
    b2d = b.reshape(1, D)
    zero = jnp.zeros((NR, D), jnp.float32)

    degpart = _deg_kernel(dst2d)
    degt = degpart.T
    g = _mm_kernel(x, W, b2d, degt)
    part = _scatter_kernel(g, zero, src3, dst3)
    out = _fin_kernel(part, degt)
    return out


# speedup vs baseline: 3.6272x; 1.1376x over previous
import functools

import jax
import jax.numpy as jnp
from jax import lax
from jax.experimental import pallas as pl
from jax.experimental.pallas import tpu as pltpu
from jax.experimental.pallas import tpu_sc as plsc

N = 10000
E = 320000
D = 128

NC = 2
NS = 16
L = 16
NW = NC * NS
EPT = E // NW
EPTP = EPT
C = 40
CH = EPTP // C
NBUF = 5
NPAD = 10240
SL = NPAD // NS
NR = 10240
RPT = NR // NS


def _deg_body(dst_hbm, degpart_hbm, dstv, dtab, blk, outv, stg):
    cid = lax.axis_index("c")
    sid = lax.axis_index("s")
    wid = cid * NS + sid

    def zero_body(i, _):
        dtab[pl.ds(i * L, L)] = jnp.zeros((L,), jnp.float32)
        return 0

    lax.fori_loop(0, NPAD // L, zero_body, 0)

    pltpu.sync_copy(dst_hbm.at[wid], dstv)

    ones = jnp.ones((L,), jnp.float32)

    def acc_body(i, _):
        idx = dstv[pl.ds(i * L, L)]
        plsc.addupdate_scatter(dtab, [idx], ones)
        return 0

    lax.fori_loop(0, EPT // L, acc_body, 0)

    pltpu.sync_copy(dtab, stg.at[sid])
    plsc.subcore_barrier()
    for t in range(NS):
        pltpu.sync_copy(stg.at[t, pl.ds(sid * SL, SL)], blk.at[t])

    def red_body(k, _):
        v = blk[0, pl.ds(k * L, L)]
        for t in range(1, NS):
            v = v + blk[t, pl.ds(k * L, L)]
        outv[pl.ds(k * L, L)] = v
        return 0

    lax.fori_loop(0, SL // L, red_body, 0)
    pltpu.sync_copy(outv, degpart_hbm.at[cid, pl.ds(sid * SL, SL)])


def _deg_kernel(dst2d):
    mesh = plsc.VectorSubcoreMesh(core_axis_name="c", subcore_axis_name="s",
                                  num_cores=NC, num_subcores=NS)
    return pl.kernel(
        _deg_body,
        out_type=jax.ShapeDtypeStruct((NC, NPAD), jnp.float32),
        mesh=mesh,
        compiler_params=pltpu.CompilerParams(needs_layout_passes=False),
        scratch_types=[
            pltpu.VMEM((EPT,), jnp.int32),
            pltpu.VMEM((NPAD,), jnp.float32),
            pltpu.VMEM((NS, SL), jnp.float32),
            pltpu.VMEM((SL,), jnp.float32),
            pltpu.VMEM_SHARED((NS, NPAD), jnp.float32),
        ],
    )(dst2d)


MMBR = 2000


def _mm_body(x_ref, w_ref, b_ref, degt_ref, g_ref):
    h = jnp.dot(x_ref[...], w_ref[...].T, preferred_element_type=jnp.float32)
    h = h + b_ref[...]
    deg = degt_ref[:, 0] + degt_ref[:, 1] + 1.0
    norm = lax.rsqrt(deg)
    g_ref[...] = h * norm[:, None]


def _mm_kernel(x, w, b2d, degt):
    grid = (N // MMBR,)
    return pl.pallas_call(
        _mm_body,
        grid=grid,
        in_specs=[
            pl.BlockSpec((MMBR, D), lambda i: (i, 0)),
            pl.BlockSpec((D, D), lambda i: (0, 0)),
            pl.BlockSpec((1, D), lambda i: (0, 0)),
            pl.BlockSpec((MMBR, NC), lambda i: (i, 0)),
        ],
        out_specs=pl.BlockSpec((MMBR, D), lambda i: (i, 0)),
        out_shape=jax.ShapeDtypeStruct((NR, D), jnp.float32),
    )(x, w, b2d, degt)


def _scatter_body(g_hbm, src_hbm, dst_hbm, part_hbm,
                  sidx, db0, db1, db2, db3, db4, rw0, rw1, rw2, rw3, rw4, acc,
                  sg0, sg1, sg2, sg3, sg4, sd0, sd1, sd2, sd3, sd4):
    cid = lax.axis_index("c")
    sid = lax.axis_index("s")
    wid = cid * NS + sid
    rsl = pl.ds(sid * RPT, RPT)
    dbufs = (db0, db1, db2, db3, db4)
    rows = (rw0, rw1, rw2, rw3, rw4)
    gsems = (sg0, sg1, sg2, sg3, sg4)
    dsems = (sd0, sd1, sd2, sd3, sd4)

    @pl.when(cid == 0)
    def _():
        pltpu.sync_copy(g_hbm.at[rsl], acc.at[rsl])

    @pl.when(cid == 1)
    def _():
        def zb(i, _):
            rw0[i // (D // L), pl.ds((i % (D // L)) * L, L)] = (
                jnp.zeros((L,), jnp.float32))
            return 0

        lax.fori_loop(0, C * (D // L), zb, 0)
        for t in range(RPT // C):
            pltpu.sync_copy(rw0, acc.at[pl.ds(sid * RPT + t * C, C)])

    pltpu.sync_copy(src_hbm.at[wid], sidx)

    def gather(j, k):
        return pltpu.make_async_copy(
            g_hbm.at[sidx.at[pl.ds(j * C, C)]], rows[k], gsems[k])

    def dfetch(j, k):
        return pltpu.make_async_copy(dst_hbm.at[wid, j], dbufs[k], dsems[k])

    for k in range(NBUF):
        dfetch(k, k).start()
        gather(k, k).start()
    plsc.subcore_barrier()

    def ring_body(q, _):
        j = q * NBUF
        for k in range(NBUF):
            jk = j + k
            gather(jk, k).wait()
            dfetch(jk, k).wait()
            pltpu.sync_copy(rows[k], acc.at[dbufs[k]], add=True)

            @pl.when(jk + NBUF < CH)
            def _():
                gather(jk + NBUF, k).start()
                dfetch(jk + NBUF, k).start()
        return 0

    lax.fori_loop(0, CH // NBUF, ring_body, 0)

    plsc.subcore_barrier()
    pltpu.sync_copy(acc.at[rsl], part_hbm.at[cid, rsl])


def _scatter_kernel(g, src3, dst3):
    mesh = plsc.VectorSubcoreMesh(core_axis_name="c", subcore_axis_name="s",
                                  num_cores=NC, num_subcores=NS)
    return pl.kernel(
        _scatter_body,
        out_type=jax.ShapeDtypeStruct((NC, NR, D), jnp.float32),
        mesh=mesh,
        scratch_types=(
            [pltpu.VMEM((EPTP,), jnp.int32)]
            + [pltpu.VMEM((C,), jnp.int32) for _ in range(NBUF)]
            + [pltpu.VMEM((C, D), jnp.float32) for _ in range(NBUF)]
            + [pltpu.VMEM_SHARED((NR, D), jnp.float32)]
            + [pltpu.SemaphoreType.DMA for _ in range(2 * NBUF)]
        ),
    )(g, src3, dst3)


def _fin_body(part_ref, degt_ref, out_ref):
    deg = degt_ref[:, 0] + degt_ref[:, 1] + 1.0
    norm = lax.rsqrt(deg)
    p = part_ref[0] + part_ref[1]
    out_ref[...] = p * norm[:, None]


def _fin_kernel(part, degt):
    grid = (N // MMBR,)
    return pl.pallas_call(
        _fin_body,
        grid=grid,
        in_specs=[
            pl.BlockSpec((NC, MMBR, D), lambda i: (0, i, 0)),
            pl.BlockSpec((MMBR, NC), lambda i: (i, 0)),
        ],
        out_specs=pl.BlockSpec((MMBR, D), lambda i: (i, 0)),
        out_shape=jax.ShapeDtypeStruct((N, D), jnp.float32),
    )(part, degt)


@jax.jit
def kernel(x, edge_index, W, b):
    src = edge_index[0]
    dst = edge_index[1]
    dst2d = dst.reshape(NW, EPT)
    src3 = src.reshape(NW, EPT)
    dst3 = dst.reshape(NW, CH, C)
    b2d = b.reshape(1, D)

    degpart = _deg_kernel(dst2d)
    degt = degpart.T
    g = _mm_kernel(x, W, b2d, degt)
    part = _scatter_kernel(g, src3, dst3)
    out = _fin_kernel(part, degt)
    return out
